# baseline (speedup 1.0000x reference)
import jax
from jax.experimental import pallas as pl
from jax.experimental.pallas import tpu as pltpu


_COLS = 2048
_TAIL = 320
_ROWS = 2008
_MAIN = 16384 - _TAIL


def _blk_copy(x_blk, o_blk):
    o_blk[...] = x_blk[...]


def _copy_kernel(x_ref, o_ref, tail_sem):
    tail = pltpu.make_async_copy(
        x_ref.at[pl.ds(_MAIN, _TAIL)],
        o_ref.at[pl.ds(_MAIN, _TAIL)],
        tail_sem,
    )
    tail.start()
    pltpu.emit_pipeline(
        _blk_copy,
        grid=(_MAIN // _ROWS,),
        in_specs=[pl.BlockSpec((_ROWS, _COLS), lambda i: (i, 0))],
        out_specs=[pl.BlockSpec((_ROWS, _COLS), lambda i: (i, 0))],
    )(x_ref.at[pl.ds(0, _MAIN)], o_ref.at[pl.ds(0, _MAIN)])
    tail.wait()


def kernel(x, s):
    total_rows = x.shape[0] * x.shape[1]
    x2 = x.reshape(total_rows, x.shape[2])
    out = pl.pallas_call(
        _copy_kernel,
        in_specs=[pl.BlockSpec(memory_space=pl.ANY)],
        out_specs=pl.BlockSpec(memory_space=pl.ANY),
        out_shape=jax.ShapeDtypeStruct(x2.shape, x.dtype),
        scratch_shapes=[pltpu.SemaphoreType.DMA],
        compiler_params=pltpu.CompilerParams(
            vmem_limit_bytes=100 * 1024 * 1024,
        ),
    )(x2)
    return out.reshape(x.shape)


# speedup vs baseline: 1.0107x; 1.0107x over previous
import jax
from jax.experimental import pallas as pl
from jax.experimental.pallas import tpu as pltpu


_COLS = 2048
_TAIL = 320
_ROWS = 2008
_MAIN = 16384 - _TAIL


def _blk_copy(x_blk, o_blk):
    o_blk[...] = x_blk[...]


def _copy_kernel(x_ref, o_ref, tail_sem):
    del tail_sem
    pltpu.emit_pipeline(
        _blk_copy,
        grid=(pl.cdiv(16384, _ROWS),),
        in_specs=[pl.BlockSpec((_ROWS, _COLS), lambda i: (i, 0))],
        out_specs=[pl.BlockSpec((_ROWS, _COLS), lambda i: (i, 0))],
    )(x_ref, o_ref)


def kernel(x, s):
    total_rows = x.shape[0] * x.shape[1]
    x2 = x.reshape(total_rows, x.shape[2])
    out = pl.pallas_call(
        _copy_kernel,
        in_specs=[pl.BlockSpec(memory_space=pl.ANY)],
        out_specs=pl.BlockSpec(memory_space=pl.ANY),
        out_shape=jax.ShapeDtypeStruct(x2.shape, x.dtype),
        scratch_shapes=[pltpu.SemaphoreType.DMA],
        compiler_params=pltpu.CompilerParams(
            vmem_limit_bytes=100 * 1024 * 1024,
        ),
    )(x2)
    return out.reshape(x.shape)
